# baseline (speedup 1.0000x reference)
import functools

import jax
import jax.numpy as jnp
from jax import lax
from jax.experimental import pallas as pl
from jax.experimental.pallas import tpu as pltpu
from jax.experimental.pallas import tpu_sc as plsc

D = 256
DE = 32
NC, NS, LANES = 2, 16, 16
NW = NC * NS


def _fold_tables_body(ep, ef, ed, wl, wi, wf, bf, bl, bi, tp_ref, tfd_ref, vli_ref):
    wfus = wf[...]
    w_p = wfus[0:32, :]
    w_l = wfus[32:64, :]
    w_f = wfus[64:96, :]
    w_i = wfus[96:128, :]
    w_d = wfus[128:136, :]
    f32 = jnp.float32
    tp_ref[...] = jnp.dot(ep[...], w_p, preferred_element_type=f32)
    bias_c = (
        bf[...]
        + jnp.dot(bl[...].reshape(1, DE), w_l, preferred_element_type=f32)[0]
        + jnp.dot(bi[...].reshape(1, DE), w_i, preferred_element_type=f32)[0]
    )
    e_f = jnp.dot(ef[...], w_f, preferred_element_type=f32)
    e_d = jnp.dot(ed[...], w_d, preferred_element_type=f32)
    tfd_ref[...] = e_f[None, :, :] + e_d[:, None, :] + bias_c[None, None, :]
    v_l = jnp.dot(wl[...], w_l, preferred_element_type=f32)
    v_i = jnp.dot(wi[...], w_i, preferred_element_type=f32)
    vli_ref[...] = jnp.concatenate([v_l, v_i], axis=0)


def _fold_tables(ep, ef, ed, wl, wi, wf, bf, bl, bi):
    return pl.pallas_call(
        _fold_tables_body,
        out_shape=[
            jax.ShapeDtypeStruct((256, D), jnp.float32),
            jax.ShapeDtypeStruct((2, 64, D), jnp.float32),
            jax.ShapeDtypeStruct((2, D), jnp.float32),
        ],
    )(ep, ef, ed, wl, wi, wf, bf, bl, bi)


def _rsqrt_nr(a):
    half = a * 0.5
    i = plsc.bitcast(a, jnp.int32)
    i = 0x5F3759DF - lax.shift_right_arithmetic(i, 1)
    y = plsc.bitcast(i, jnp.float32)
    for _ in range(3):
        y = y * (1.5 - half * y * y)
    return y


def _make_sc_kernel(n_rows):
    PW = n_rows // NW
    CH = 1600
    NCHUNK = PW // CH
    TPC = CH // LANES
    mesh = plsc.VectorSubcoreMesh(core_axis_name="c", subcore_axis_name="s")

    @functools.partial(
        pl.kernel,
        out_type=jax.ShapeDtypeStruct((n_rows, D), jnp.float32),
        mesh=mesh,
        compiler_params=pltpu.CompilerParams(
            use_tc_tiling_on_sc=False, needs_layout_passes=False),
        scratch_types=[
            pltpu.VMEM((256, D), jnp.float32),
            pltpu.VMEM((128, D), jnp.float32),
            pltpu.VMEM((2, D), jnp.float32),
            pltpu.VMEM((2, D), jnp.float32),
            pltpu.VMEM((CH,), jnp.float32),
            pltpu.VMEM((CH,), jnp.float32),
            pltpu.VMEM((CH,), jnp.float32),
            pltpu.VMEM((CH,), jnp.float32),
            pltpu.VMEM((CH,), jnp.float32),
            pltpu.VMEM((LANES * D,), jnp.float32),
            pltpu.VMEM((LANES, D), jnp.float32),
        ],
    )
    def sc_kernel(tp_h, tfd_h, vli_h, gb_h, xp_h, xl_h, xf_h, xi_h, xd_h,
                  out_h, tpv, tfdv, vliv, gbv, bufp, bufl, buff, bufi, bufd,
                  hbuf, obuf):
        wid = lax.axis_index("s") * NC + lax.axis_index("c")
        base = wid * PW
        pltpu.sync_copy(tp_h, tpv)
        pltpu.sync_copy(tfd_h, tfdv)
        pltpu.sync_copy(vli_h, vliv)
        pltpu.sync_copy(gb_h, gbv)

        zv = jnp.zeros((LANES,), jnp.int32)
        ov = zv + 1
        lanes = lax.iota(jnp.int32, LANES)
        inv_d = jnp.float32(1.0 / D)

        def chunk_body(ci, _):
            cb = base + ci * CH
            pltpu.sync_copy(xp_h.at[pl.ds(cb, CH)], bufp)
            pltpu.sync_copy(xl_h.at[pl.ds(cb, CH)], bufl)
            pltpu.sync_copy(xf_h.at[pl.ds(cb, CH)], buff)
            pltpu.sync_copy(xi_h.at[pl.ds(cb, CH)], bufi)
            pltpu.sync_copy(xd_h.at[pl.ds(cb, CH)], bufd)

            def tile_body(t, tcarry):
                t16 = t * LANES
                p16 = jnp.clip(bufp[pl.ds(t16, LANES)].astype(jnp.int32), 0, 255)
                f16 = jnp.clip(buff[pl.ds(t16, LANES)].astype(jnp.int32), 0, 63)
                d16 = jnp.clip(bufd[pl.ds(t16, LANES)].astype(jnp.int32), 0, 1)
                fd16 = d16 * 64 + f16
                len16 = bufl[pl.ds(t16, LANES)]
                iat16 = bufi[pl.ds(t16, LANES)]

                def col_body(c, carry):
                    colv, s, q = carry
                    gp = plsc.load_gather(tpv, [p16, colv])
                    gfd = plsc.load_gather(tfdv, [fd16, colv])
                    vl = plsc.load_gather(vliv, [zv, colv])
                    vi = plsc.load_gather(vliv, [ov, colv])
                    h = gp + gfd + len16 * vl + iat16 * vi
                    hbuf[pl.ds(c * LANES, LANES)] = h
                    return colv + 1, s + h, q + h * h

                _cv, s, q = lax.fori_loop(
                    0, D, col_body, (zv, jnp.zeros((LANES,), jnp.float32),
                                     jnp.zeros((LANES,), jnp.float32)))
                m = s * inv_d
                var = q * inv_d - m * m
                rn = _rsqrt_nr(var + 1e-5)

                def col_norm(c, colv):
                    h = hbuf[pl.ds(c * LANES, LANES)]
                    g = plsc.load_gather(gbv, [zv, colv])
                    b = plsc.load_gather(gbv, [ov, colv])
                    hn = (h - m) * rn * g + b
                    plsc.store_scatter(obuf, [lanes, colv], hn)
                    return colv + 1

                lax.fori_loop(0, D, col_norm, zv)
                pltpu.sync_copy(obuf, out_h.at[pl.ds(cb + t16, LANES), :])
                return tcarry

            return lax.fori_loop(0, TPC, tile_body, _)

        lax.fori_loop(0, NCHUNK, chunk_body, 0)

    return sc_kernel


def kernel(x, emb_proto, emb_flags, emb_dir, w_len, b_len, w_iat, b_iat,
           w_fus, b_fus, gamma, beta):
    B, L, _ = x.shape
    n_rows = B * L
    tp, tfd, vli = _fold_tables(emb_proto, emb_flags, emb_dir, w_len, w_iat,
                                w_fus, b_fus, b_len, b_iat)
    tfd = tfd.reshape(128, D)
    gb = jnp.stack([gamma, beta])
    xf = x.reshape(n_rows, 5)
    cols = [xf[:, k] for k in range(5)]
    out = _make_sc_kernel(n_rows)(tp, tfd, vli, gb, *cols)
    return out.reshape(B, L, D)


# speedup vs baseline: 1.1625x; 1.1625x over previous
import functools

import jax
import jax.numpy as jnp
from jax import lax
from jax.experimental import pallas as pl
from jax.experimental.pallas import tpu as pltpu
from jax.experimental.pallas import tpu_sc as plsc

D = 256
DE = 32
NC, NS, LANES = 2, 16, 16
NW = NC * NS


def _fold_tables_body(ep, ef, ed, wl, wi, wf, bf, bl, bi, tp_ref, tfd_ref, vli_ref):
    wfus = wf[...]
    w_p = wfus[0:32, :]
    w_l = wfus[32:64, :]
    w_f = wfus[64:96, :]
    w_i = wfus[96:128, :]
    w_d = wfus[128:136, :]
    f32 = jnp.float32
    tp_ref[...] = jnp.dot(ep[...], w_p, preferred_element_type=f32)
    bias_c = (
        bf[...]
        + jnp.dot(bl[...].reshape(1, DE), w_l, preferred_element_type=f32)[0]
        + jnp.dot(bi[...].reshape(1, DE), w_i, preferred_element_type=f32)[0]
    )
    e_f = jnp.dot(ef[...], w_f, preferred_element_type=f32)
    e_d = jnp.dot(ed[...], w_d, preferred_element_type=f32)
    tfd_ref[...] = e_f[None, :, :] + e_d[:, None, :] + bias_c[None, None, :]
    v_l = jnp.dot(wl[...], w_l, preferred_element_type=f32)
    v_i = jnp.dot(wi[...], w_i, preferred_element_type=f32)
    vli_ref[...] = jnp.concatenate([v_l, v_i], axis=0)


def _fold_tables(ep, ef, ed, wl, wi, wf, bf, bl, bi):
    return pl.pallas_call(
        _fold_tables_body,
        out_shape=[
            jax.ShapeDtypeStruct((256, D), jnp.float32),
            jax.ShapeDtypeStruct((2, 64, D), jnp.float32),
            jax.ShapeDtypeStruct((2, D), jnp.float32),
        ],
    )(ep, ef, ed, wl, wi, wf, bf, bl, bi)


def _rsqrt_nr(a):
    half = a * 0.5
    i = plsc.bitcast(a, jnp.int32)
    i = 0x5F3759DF - lax.shift_right_arithmetic(i, 1)
    y = plsc.bitcast(i, jnp.float32)
    for _ in range(3):
        y = y * (1.5 - half * y * y)
    return y


def _make_sc_kernel(n_rows):
    PW = n_rows // NW
    CH = 1600
    NCHUNK = PW // CH
    PPC = CH // (2 * LANES)
    UN = 4
    mesh = plsc.VectorSubcoreMesh(core_axis_name="c", subcore_axis_name="s")

    @functools.partial(
        pl.kernel,
        out_type=jax.ShapeDtypeStruct((n_rows, D), jnp.float32),
        mesh=mesh,
        compiler_params=pltpu.CompilerParams(
            use_tc_tiling_on_sc=False, needs_layout_passes=False),
        scratch_types=[
            pltpu.VMEM((256, D), jnp.float32),
            pltpu.VMEM((128, D), jnp.float32),
            pltpu.VMEM((2, D), jnp.float32),
            pltpu.VMEM((2, D), jnp.float32),
            pltpu.VMEM((CH,), jnp.float32),
            pltpu.VMEM((CH,), jnp.float32),
            pltpu.VMEM((CH,), jnp.float32),
            pltpu.VMEM((CH,), jnp.float32),
            pltpu.VMEM((CH,), jnp.float32),
            pltpu.VMEM((4 * LANES, D), jnp.float32),
            pltpu.SemaphoreType.DMA((2,)),
        ],
    )
    def sc_kernel(tp_h, tfd_h, vli_h, gb_h, xp_h, xl_h, xf_h, xi_h, xd_h,
                  out_h, tpv, tfdv, vliv, gbv, bufp, bufl, buff, bufi, bufd,
                  obuf, sem):
        wid = lax.axis_index("s") * NC + lax.axis_index("c")
        base = wid * PW
        pltpu.sync_copy(tp_h, tpv)
        pltpu.sync_copy(tfd_h, tfdv)
        pltpu.sync_copy(vli_h, vliv)
        pltpu.sync_copy(gb_h, gbv)

        zv = jnp.zeros((LANES,), jnp.int32)
        lanes = lax.iota(jnp.int32, LANES)
        inv_d = jnp.float32(1.0 / D)
        fz = jnp.zeros((LANES,), jnp.float32)

        def load_idx(buf, t0):
            p16 = jnp.clip(bufp[pl.ds(t0, LANES)].astype(jnp.int32), 0, 255)
            f16 = jnp.clip(buff[pl.ds(t0, LANES)].astype(jnp.int32), 0, 63)
            d16 = jnp.clip(bufd[pl.ds(t0, LANES)].astype(jnp.int32), 0, 1)
            return p16, d16 * 64 + f16, bufl[pl.ds(t0, LANES)], bufi[pl.ds(t0, LANES)]

        def chunk_body(ci, _):
            cb = base + ci * CH
            pltpu.sync_copy(xp_h.at[pl.ds(cb, CH)], bufp)
            pltpu.sync_copy(xl_h.at[pl.ds(cb, CH)], bufl)
            pltpu.sync_copy(xf_h.at[pl.ds(cb, CH)], buff)
            pltpu.sync_copy(xi_h.at[pl.ds(cb, CH)], bufi)
            pltpu.sync_copy(xd_h.at[pl.ds(cb, CH)], bufd)

            def pair_body(pj, _pc):
                gp_idx = ci * PPC + pj
                buf = lax.rem(gp_idx, 2)
                t0 = pj * (2 * LANES)
                rowb = cb + t0

                @pl.when(gp_idx >= 2)
                def _():
                    pltpu.make_async_copy(
                        obuf.at[pl.ds(buf * (2 * LANES), 2 * LANES), :],
                        out_h.at[pl.ds(base, 2 * LANES), :],
                        sem.at[buf]).wait()

                pA = load_idx(buf, t0)
                pB = load_idx(buf, t0 + LANES)
                rowsA = lanes + buf * (2 * LANES)
                rowsB = rowsA + LANES

                def col_body(c, carry):
                    sA, qA, sB, qB = carry
                    colv = zv + c
                    vl = plsc.load_gather(vliv, [zv, colv])
                    vi = plsc.load_gather(vliv, [zv + 1, colv])
                    gpA = plsc.load_gather(tpv, [pA[0], colv])
                    gfA = plsc.load_gather(tfdv, [pA[1], colv])
                    gpB = plsc.load_gather(tpv, [pB[0], colv])
                    gfB = plsc.load_gather(tfdv, [pB[1], colv])
                    hA = gpA + gfA + pA[2] * vl + pA[3] * vi
                    hB = gpB + gfB + pB[2] * vl + pB[3] * vi
                    plsc.store_scatter(obuf, [rowsA, colv], hA)
                    plsc.store_scatter(obuf, [rowsB, colv], hB)
                    return sA + hA, qA + hA * hA, sB + hB, qB + hB * hB

                sA, qA, sB, qB = plsc.parallel_loop(
                    0, D, unroll=UN, carry=(fz, fz, fz, fz))(col_body)
                mA = sA * inv_d
                mB = sB * inv_d
                rA = _rsqrt_nr(qA * inv_d - mA * mA + 1e-5)
                rB = _rsqrt_nr(qB * inv_d - mB * mB + 1e-5)

                def col_norm(c):
                    colv = zv + c
                    g = plsc.load_gather(gbv, [zv, colv])
                    b = plsc.load_gather(gbv, [zv + 1, colv])
                    hA = plsc.load_gather(obuf, [rowsA, colv])
                    hB = plsc.load_gather(obuf, [rowsB, colv])
                    hnA = (hA - mA) * rA * g + b
                    hnB = (hB - mB) * rB * g + b
                    plsc.store_scatter(obuf, [rowsA, colv], hnA)
                    plsc.store_scatter(obuf, [rowsB, colv], hnB)

                plsc.parallel_loop(0, D, unroll=UN)(col_norm)

                pltpu.async_copy(
                    obuf.at[pl.ds(buf * (2 * LANES), 2 * LANES), :],
                    out_h.at[pl.ds(rowb, 2 * LANES), :],
                    sem.at[buf])
                return _pc

            return lax.fori_loop(0, PPC, pair_body, _)

        lax.fori_loop(0, NCHUNK, chunk_body, 0)

        for b in range(2):
            pltpu.make_async_copy(
                obuf.at[pl.ds(b * (2 * LANES), 2 * LANES), :],
                out_h.at[pl.ds(base, 2 * LANES), :],
                sem.at[b]).wait()

    return sc_kernel


def kernel(x, emb_proto, emb_flags, emb_dir, w_len, b_len, w_iat, b_iat,
           w_fus, b_fus, gamma, beta):
    B, L, _ = x.shape
    n_rows = B * L
    tp, tfd, vli = _fold_tables(emb_proto, emb_flags, emb_dir, w_len, w_iat,
                                w_fus, b_fus, b_len, b_iat)
    tfd = tfd.reshape(128, D)
    gb = jnp.stack([gamma, beta])
    xf = x.reshape(n_rows, 5)
    cols = [xf[:, k] for k in range(5)]
    out = _make_sc_kernel(n_rows)(tp, tfd, vli, gb, *cols)
    return out.reshape(B, L, D)
